# baseline (speedup 1.0000x reference)
import jax
import jax.numpy as jnp
from jax.experimental import pallas as pl
from jax.experimental.pallas import tpu as pltpu

_GRP = 48


def _tc_body(labels_ref, wids_ref, pts_ref, tf_ref, out_ref):
    i = jax.lax.broadcasted_iota(jnp.int32, (_GRP, _GRP), 0)
    j = jax.lax.broadcasted_iota(jnp.int32, (_GRP, _GRP), 1)
    same_blk = (i // 3) == (j // 3)
    a_comp = i % 3
    c_comp = j % 3
    A = jnp.zeros((_GRP, _GRP), jnp.float32)
    for c0 in range(3):
        for a0 in range(3):
            r = tf_ref[0, 0, 4 * c0 + a0]
            A = A + jnp.where(same_blk & (a_comp == a0) & (c_comp == c0), r, 0.0)
    jrow = jax.lax.broadcasted_iota(jnp.int32, (1, _GRP), 1)
    T = jnp.zeros((1, _GRP), jnp.float32)
    for c0 in range(3):
        T = T + jnp.where(jrow % 3 == c0, tf_ref[0, 0, 4 * c0 + 3], 0.0)
    p = pts_ref[0]
    out_ref[0] = jnp.dot(p, A, preferred_element_type=jnp.float32) + T


def kernel(pts_table, transforms, label_ids, write_ids):
    num_labels, n_points, _ = pts_table.shape
    bsz = transforms.shape[0]
    rows = n_points * 3 // _GRP
    pts_flat = pts_table.reshape(num_labels, rows, _GRP)
    tf_flat = transforms.reshape(bsz, 1, 16)

    grid_spec = pltpu.PrefetchScalarGridSpec(
        num_scalar_prefetch=2,
        grid=(bsz,),
        in_specs=[
            pl.BlockSpec((1, rows, _GRP), lambda b, lbl, wid: (lbl[b], 0, 0)),
            pl.BlockSpec((1, 1, 16), lambda b, lbl, wid: (b, 0, 0)),
        ],
        out_specs=pl.BlockSpec((1, rows, _GRP), lambda b, lbl, wid: (wid[b], 0, 0)),
    )
    out = pl.pallas_call(
        _tc_body,
        grid_spec=grid_spec,
        out_shape=jax.ShapeDtypeStruct((bsz, rows, _GRP), jnp.float32),
    )(label_ids.astype(jnp.int32), write_ids.astype(jnp.int32), pts_flat, tf_flat)
    return out.reshape(bsz, n_points, 3)


# speedup vs baseline: 1.1067x; 1.1067x over previous
import jax
import jax.numpy as jnp
from jax.experimental import pallas as pl
from jax.experimental.pallas import tpu as pltpu

_LANES = 240
_ROWS = 250


def _tc_body(labels_ref, wids_ref, pts_ref, tf_ref, out_ref):
    c = jax.lax.broadcasted_iota(jnp.int32, (1, _LANES), 1) % 3
    p = pts_ref[0]
    acc = jnp.zeros((1, _LANES), jnp.float32)
    for c0 in range(3):
        acc = acc + jnp.where(c == c0, tf_ref[0, 0, 4 * c0 + 3], 0.0)
    out = jnp.broadcast_to(acc, p.shape)
    for d in range(-2, 3):
        w = jnp.zeros((1, _LANES), jnp.float32)
        for c0 in range(3):
            j0 = c0 + d
            if 0 <= j0 < 3:
                w = w + jnp.where(c == c0, tf_ref[0, 0, 4 * c0 + j0], 0.0)
        shifted = pltpu.roll(p, (-d) % _LANES, 1)
        out = out + w * shifted
    out_ref[0] = out


def kernel(pts_table, transforms, label_ids, write_ids):
    num_labels, n_points, _ = pts_table.shape
    bsz = transforms.shape[0]
    rows = n_points * 3 // _LANES
    pts_flat = pts_table.reshape(num_labels, rows, _LANES)
    tf_flat = transforms.reshape(bsz, 1, 16)

    grid_spec = pltpu.PrefetchScalarGridSpec(
        num_scalar_prefetch=2,
        grid=(bsz,),
        in_specs=[
            pl.BlockSpec((1, rows, _LANES), lambda b, lbl, wid: (lbl[b], 0, 0)),
            pl.BlockSpec((1, 1, 16), lambda b, lbl, wid: (b, 0, 0)),
        ],
        out_specs=pl.BlockSpec((1, rows, _LANES), lambda b, lbl, wid: (wid[b], 0, 0)),
    )
    out = pl.pallas_call(
        _tc_body,
        grid_spec=grid_spec,
        out_shape=jax.ShapeDtypeStruct((bsz, rows, _LANES), jnp.float32),
    )(label_ids.astype(jnp.int32), write_ids.astype(jnp.int32), pts_flat, tf_flat)
    return out.reshape(bsz, n_points, 3)


# speedup vs baseline: 12.4697x; 11.2673x over previous
import jax
import jax.numpy as jnp
from jax import lax
from jax.experimental import pallas as pl
from jax.experimental.pallas import tpu as pltpu

_PC = 1000


def _body(pts_ref, tf_ref, lbl_ref, out_ref):
    c = pl.program_id(0)
    nl = 64
    bsz = lbl_ref.shape[1]
    tf_rows = tf_ref[...]
    lbl = lbl_ref[...]

    def tfrow(j):
        r0 = tf_rows[j:j + 1, :]
        r1 = tf_rows[4 + j:5 + j, :]
        r2 = tf_rows[8 + j:9 + j, :]
        return jnp.where(c == 0, r0, jnp.where(c == 1, r1, r2))

    lrow = lax.broadcasted_iota(jnp.int32, (3 * nl, bsz), 0) % nl
    onehot = (lrow == jnp.broadcast_to(lbl, (3 * nl, bsz))).astype(jnp.float32)
    coeff = jnp.concatenate(
        [jnp.broadcast_to(tfrow(j), (nl, bsz)) for j in range(3)], axis=0)
    W = onehot * coeff
    t_row = tfrow(3)
    p = pts_ref[...]
    acc = lax.dot_general(
        p, W, (((1,), (0,)), ((), ())), preferred_element_type=jnp.float32)
    out_ref[...] = acc + t_row


def kernel(pts_table, transforms, label_ids, write_ids):
    num_labels, n_points, _ = pts_table.shape
    bsz = transforms.shape[0]
    pts2 = pts_table.transpose(2, 0, 1).reshape(3 * num_labels, n_points)
    pts_t = pts2.T
    inv = jnp.zeros((bsz,), jnp.int32).at[write_ids.astype(jnp.int32)].set(
        jnp.arange(bsz, dtype=jnp.int32))
    tfT = transforms.reshape(bsz, 16).T[:, inv]
    lblT = label_ids.astype(jnp.int32)[inv][None, :]

    pc = _PC if n_points % _PC == 0 else n_points
    nch = n_points // pc
    out2 = pl.pallas_call(
        _body,
        grid=(3, nch),
        in_specs=[
            pl.BlockSpec((pc, 3 * num_labels), lambda c, j: (j, 0)),
            pl.BlockSpec((16, bsz), lambda c, j: (0, 0)),
            pl.BlockSpec((1, bsz), lambda c, j: (0, 0)),
        ],
        out_specs=pl.BlockSpec((pc, bsz), lambda c, j: (c * nch + j, 0)),
        out_shape=jax.ShapeDtypeStruct((3 * n_points, bsz), jnp.float32),
    )(pts_t, tfT, lblT)
    return out2.reshape(3, n_points, bsz).transpose(2, 1, 0)


# speedup vs baseline: 20.1512x; 1.6160x over previous
import jax
import jax.numpy as jnp
from jax import lax
from jax.experimental import pallas as pl
from jax.experimental.pallas import tpu as pltpu

_BC = 128


def _body(pts_ref, tf_ref, lbl_ref, out_ref):
    c = pl.program_id(0)
    nl = 64
    bsz = lbl_ref.shape[1]
    tf_rows = tf_ref[...]
    lbl = lbl_ref[...]

    def tfrow(j):
        r0 = tf_rows[j:j + 1, :]
        r1 = tf_rows[4 + j:5 + j, :]
        r2 = tf_rows[8 + j:9 + j, :]
        return jnp.where(c == 0, r0, jnp.where(c == 1, r1, r2))

    lrow = lax.broadcasted_iota(jnp.int32, (3 * nl, bsz), 0) % nl
    onehot = (lrow == jnp.broadcast_to(lbl, (3 * nl, bsz))).astype(jnp.float32)
    coeff = jnp.concatenate(
        [jnp.broadcast_to(tfrow(j), (nl, bsz)) for j in range(3)], axis=0)
    W = onehot * coeff
    t_row = tfrow(3)
    p = pts_ref[...]
    acc = lax.dot_general(
        p, W, (((0,), (0,)), ((), ())), preferred_element_type=jnp.float32)
    out_ref[...] = acc + t_row


def kernel(pts_table, transforms, label_ids, write_ids):
    num_labels, n_points, _ = pts_table.shape
    bsz = transforms.shape[0]
    pts2 = pts_table.transpose(2, 0, 1).reshape(3 * num_labels, n_points)
    inv = jnp.zeros((bsz,), jnp.int32).at[write_ids.astype(jnp.int32)].set(
        jnp.arange(bsz, dtype=jnp.int32))
    tfT = transforms.reshape(bsz, 16).T[:, inv]
    lblT = label_ids.astype(jnp.int32)[inv][None, :]

    bc = _BC if bsz % _BC == 0 else bsz
    nbc = bsz // bc
    out2 = pl.pallas_call(
        _body,
        grid=(3, nbc),
        in_specs=[
            pl.BlockSpec((3 * num_labels, n_points), lambda c, j: (0, 0)),
            pl.BlockSpec((16, bc), lambda c, j: (0, j)),
            pl.BlockSpec((1, bc), lambda c, j: (0, j)),
        ],
        out_specs=pl.BlockSpec((n_points, bc), lambda c, j: (c, j)),
        out_shape=jax.ShapeDtypeStruct((3 * n_points, bsz), jnp.float32),
    )(pts2, tfT, lblT)
    return out2.reshape(3, n_points, bsz).transpose(2, 1, 0)


# speedup vs baseline: 25.5623x; 1.2685x over previous
import jax
import jax.numpy as jnp
from jax import lax
from jax.experimental import pallas as pl
from jax.experimental.pallas import tpu as pltpu

_BC = 256


def _body(pts_ref, tf_ref, lbl_ref, out_ref):
    c = pl.program_id(0)
    nl = pts_ref.shape[0] // 3
    bsz = lbl_ref.shape[1]
    tf_rows = tf_ref[...]
    lbl = lbl_ref[...]

    def tfrow(j):
        r0 = tf_rows[j:j + 1, :]
        r1 = tf_rows[4 + j:5 + j, :]
        r2 = tf_rows[8 + j:9 + j, :]
        return jnp.where(c == 0, r0, jnp.where(c == 1, r1, r2))

    lrow = lax.broadcasted_iota(jnp.int32, (3 * nl, bsz), 0) % nl
    onehot = (lrow == jnp.broadcast_to(lbl, (3 * nl, bsz))).astype(jnp.float32)
    coeff = jnp.concatenate(
        [jnp.broadcast_to(tfrow(j), (nl, bsz)) for j in range(3)], axis=0)
    W = onehot * coeff
    t_row = tfrow(3)
    p = pts_ref[...]
    acc = lax.dot_general(
        p, W, (((0,), (0,)), ((), ())), preferred_element_type=jnp.float32)
    out_ref[...] = acc + t_row


def kernel(pts_table, transforms, label_ids, write_ids):
    num_labels, n_points, _ = pts_table.shape
    bsz = transforms.shape[0]
    pts2 = pts_table.transpose(2, 0, 1).reshape(3 * num_labels, n_points)
    inv = jnp.zeros((bsz,), jnp.int32).at[write_ids.astype(jnp.int32)].set(
        jnp.arange(bsz, dtype=jnp.int32))
    tfT = transforms.reshape(bsz, 16).T[:, inv]
    lblT = label_ids.astype(jnp.int32)[inv][None, :]

    bc = _BC if bsz % _BC == 0 else bsz
    nbc = bsz // bc
    out2 = pl.pallas_call(
        _body,
        grid=(3, nbc),
        in_specs=[
            pl.BlockSpec((3 * num_labels, n_points), lambda c, j: (0, 0)),
            pl.BlockSpec((16, bc), lambda c, j: (0, j)),
            pl.BlockSpec((1, bc), lambda c, j: (0, j)),
        ],
        out_specs=pl.BlockSpec((n_points, bc), lambda c, j: (c, j)),
        out_shape=jax.ShapeDtypeStruct((3 * n_points, bsz), jnp.float32),
    )(pts2, tfT, lblT)
    return out2.reshape(3, n_points, bsz).transpose(2, 1, 0)


# speedup vs baseline: 25.5744x; 1.0005x over previous
import jax
import jax.numpy as jnp
from jax import lax
from jax.experimental import pallas as pl

_BC = 256


def _body(pts_ref, tf_ref, lbl_ref, out_ref):
    c = pl.program_id(0)
    nl = pts_ref.shape[0] // 3
    bsz = lbl_ref.shape[1]
    tf_rows = tf_ref[...]
    lbl = lbl_ref[...]

    def tfrow(j):
        r0 = tf_rows[j:j + 1, :]
        r1 = tf_rows[4 + j:5 + j, :]
        r2 = tf_rows[8 + j:9 + j, :]
        return jnp.where(c == 0, r0, jnp.where(c == 1, r1, r2))

    lrow = lax.broadcasted_iota(jnp.int32, (3 * nl, bsz), 0) % nl
    onehot = (lrow == jnp.broadcast_to(lbl, (3 * nl, bsz))).astype(jnp.float32)
    coeff = jnp.concatenate(
        [jnp.broadcast_to(tfrow(j), (nl, bsz)) for j in range(3)], axis=0)
    W = onehot * coeff
    t_row = tfrow(3)
    p = pts_ref[...]
    acc = lax.dot_general(
        p, W, (((0,), (0,)), ((), ())), preferred_element_type=jnp.float32)
    out_ref[...] = acc + t_row


def kernel(pts_table, transforms, label_ids, write_ids):
    num_labels, n_points, _ = pts_table.shape
    bsz = transforms.shape[0]
    pts2 = pts_table.transpose(2, 0, 1).reshape(3 * num_labels, n_points)
    inv = jnp.zeros((bsz,), jnp.int32).at[write_ids.astype(jnp.int32)].set(
        jnp.arange(bsz, dtype=jnp.int32))
    tfT = transforms.reshape(bsz, 16).T[:, inv]
    lblT = label_ids.astype(jnp.int32)[inv][None, :]

    bc = _BC if bsz % _BC == 0 else bsz
    nbc = bsz // bc
    out2 = pl.pallas_call(
        _body,
        grid=(3, nbc),
        in_specs=[
            pl.BlockSpec((3 * num_labels, n_points), lambda c, j: (0, 0)),
            pl.BlockSpec((16, bc), lambda c, j: (0, j)),
            pl.BlockSpec((1, bc), lambda c, j: (0, j)),
        ],
        out_specs=pl.BlockSpec((n_points, bc), lambda c, j: (c, j)),
        out_shape=jax.ShapeDtypeStruct((3 * n_points, bsz), jnp.float32),
    )(pts2, tfT, lblT)
    return out2.reshape(3, n_points, bsz).transpose(2, 1, 0)
